# Initial kernel scaffold; baseline (speedup 1.0000x reference)
import jax
import jax.numpy as jnp
from jax.experimental import pallas as pl


def kernel(x, edge_index_r0, edge_index_r1, edge_index_r2, W_g, b_g, W_lin, b_lin):
    raise NotImplementedError("write your pallas kernel here")


# speedup vs baseline: 1.3259x; 1.3259x over previous
import jax
import jax.numpy as jnp
from jax import lax
from jax.experimental import pallas as pl
from jax.experimental.pallas import tpu as pltpu
from jax.experimental.pallas import tpu_sc as plsc

N = 50000
D = 128
R = 3
E = 200000
L = 5

N_PAD = 51200
ROWS_PER_TILE = N_PAD // 16
E_PAD = 200704
BLK = 128
NBLK = 98
KFL = 2
GRP = NBLK // KFL
P_BKT = 8832
NBKT = 6
A_ROWS = NBKT * P_BKT
SP_ROWS = P_BKT + 128
ZTR = SP_ROWS // 16
BN = 512
GRID = N_PAD // BN

_f32 = jnp.float32


def _mesh():
    return plsc.VectorSubcoreMesh(core_axis_name="c", subcore_axis_name="s")


def _deg_body(edges, z1, deg, spd, idxb, ones_b):
    c = lax.axis_index("c")
    s = lax.axis_index("s")
    o16 = jnp.ones((16,), _f32)
    for i in range(BLK // 16):
        ones_b[pl.ds(i * 16, 16)] = o16

    for k in range(R):
        pltpu.sync_copy(z1, spd.at[pl.ds(s * ROWS_PER_TILE, ROWS_PER_TILE)])
        plsc.subcore_barrier()
        pltpu.sync_copy(edges.at[k, c, s], idxb)

        def sc_body(j, carry):
            pltpu.sync_copy(ones_b, spd.at[idxb.at[j]], add=True)
            return carry

        lax.fori_loop(0, NBLK, sc_body, 0)
        plsc.subcore_barrier()
        off = (c * R + k) * N_PAD + s * ROWS_PER_TILE
        pltpu.sync_copy(
            spd.at[pl.ds(s * ROWS_PER_TILE, ROWS_PER_TILE)],
            deg.at[pl.ds(off, ROWS_PER_TILE)],
        )
        plsc.subcore_barrier()


def _deg_call(edges, z1):
    return pl.kernel(
        _deg_body,
        out_type=jax.ShapeDtypeStruct((2 * R * N_PAD,), _f32),
        mesh=_mesh(),
        scratch_types=[
            pltpu.VMEM_SHARED((N_PAD,), _f32),
            pltpu.VMEM((NBLK, BLK), jnp.int32),
            pltpu.VMEM((BLK,), _f32),
        ],
    )(edges, z1)


def _spmv_body(h0, h1, h2, edges, z2, agg, spa, isrc, idst, bufs, gsem):
    c = lax.axis_index("c")
    s = lax.axis_index("s")
    lanes = lax.iota(jnp.int32, 16)

    for r in range(R):
        h_r = (h0, h1, h2)[r]
        pltpu.sync_copy(edges.at[r, 0, s], isrc)
        for p in range(NBKT // 2):
            bkt = 2 * p + c
            base = bkt * P_BKT
            pltpu.sync_copy(edges.at[r, 1, s], idst)

            def remap(j, carry):
                for kk in range(8):
                    v = idst[j, pl.ds(kk * 16, 16)]
                    rel = v - base
                    ok = (rel >= 0) & (rel < P_BKT)
                    idst[j, pl.ds(kk * 16, 16)] = jnp.where(
                        ok, rel, P_BKT + lanes)
                return carry

            lax.fori_loop(0, NBLK, remap, 0)

            pltpu.sync_copy(z2, spa.at[pl.ds(s * ZTR, ZTR)])
            plsc.subcore_barrier()

            def g_body(g, carry):
                hs = []
                for b in range(KFL):
                    j = g * KFL + b
                    hs.append(pltpu.async_copy(
                        h_r.at[isrc.at[j]], bufs.at[b], gsem))
                for b in range(KFL):
                    hs[b].wait()
                for b in range(KFL):
                    j = g * KFL + b
                    pltpu.sync_copy(
                        bufs.at[b], spa.at[idst.at[j]], add=True)
                return carry

            lax.fori_loop(0, GRP, g_body, 0)
            plsc.subcore_barrier()
            nwb = P_BKT // 16
            pltpu.sync_copy(
                spa.at[pl.ds(s * nwb, nwb)],
                agg.at[r, pl.ds(base + s * nwb, nwb)],
            )
            plsc.subcore_barrier()


def _spmv_call(h0, h1, h2, edges, z2):
    return pl.kernel(
        _spmv_body,
        out_type=jax.ShapeDtypeStruct((R, A_ROWS, D), _f32),
        mesh=_mesh(),
        scratch_types=[
            pltpu.VMEM_SHARED((SP_ROWS, D), _f32),
            pltpu.VMEM((NBLK, BLK), jnp.int32),
            pltpu.VMEM((NBLK, BLK), jnp.int32),
            pltpu.VMEM((KFL, BLK, D), _f32),
            pltpu.SemaphoreType.DMA,
        ],
    )(h0, h1, h2, edges, z2)


def _cs(deg_ref, r):
    return lax.rsqrt(jnp.maximum(deg_ref[0, r], 1.0))


def _cd3(deg_ref, r):
    return lax.rsqrt(jnp.maximum(deg_ref[1, r], 1.0)) * (1.0 / 3.0)


def _prologue_kernel(x_ref, deg_ref, o0, o1, o2):
    xv = x_ref[...]
    for r, o in enumerate((o0, o1, o2)):
        o[...] = xv * _cs(deg_ref, r)


def _accum(agg_ref, deg_ref, w_ref, b_ref):
    acc = jnp.broadcast_to(b_ref[...], (BN, D)).astype(_f32)
    wv = w_ref[...]
    for r in range(R):
        a = agg_ref[r] * _cd3(deg_ref, r)
        acc = acc + jnp.dot(a, wv[r], preferred_element_type=_f32)
    return acc


def _layer_kernel(agg_ref, deg_ref, w_ref, b_ref, o0, o1, o2):
    acc = _accum(agg_ref, deg_ref, w_ref, b_ref)
    h = jnp.where(acc >= 0, acc, 0.01 * acc)
    for r, o in enumerate((o0, o1, o2)):
        o[...] = h * _cs(deg_ref, r)


def _final_kernel(agg_ref, deg_ref, w_ref, b_ref, wl_ref, bl_ref, out_ref):
    h = _accum(agg_ref, deg_ref, w_ref, b_ref)
    out_ref[...] = (
        jnp.dot(h, wl_ref[...], preferred_element_type=_f32) + bl_ref[...]
    )


_DEG_SPEC = pl.BlockSpec((2, R, BN, 1), lambda i: (0, 0, i, 0))
_AGG_SPEC = pl.BlockSpec((R, BN, D), lambda i: (0, i, 0))
_W_SPEC = pl.BlockSpec((R, D, D), lambda i: (0, 0, 0))
_B_SPEC = pl.BlockSpec((1, D), lambda i: (0, 0))
_H_SPEC = pl.BlockSpec((BN, D), lambda i: (i, 0))
_H_TYPE = jax.ShapeDtypeStruct((N_PAD, D), _f32)


def _prologue_call(x_pad, deg4):
    return pl.pallas_call(
        _prologue_kernel,
        grid=(GRID,),
        in_specs=[_H_SPEC, _DEG_SPEC],
        out_specs=[_H_SPEC] * 3,
        out_shape=[_H_TYPE] * 3,
    )(x_pad, deg4)


def _layer_call(agg, deg4, w, bbar):
    return pl.pallas_call(
        _layer_kernel,
        grid=(GRID,),
        in_specs=[_AGG_SPEC, _DEG_SPEC, _W_SPEC, _B_SPEC],
        out_specs=[_H_SPEC] * 3,
        out_shape=[_H_TYPE] * 3,
    )(agg, deg4, w, bbar)


def _final_call(agg, deg4, w, bbar, w_lin, b_lin):
    return pl.pallas_call(
        _final_kernel,
        grid=(GRID,),
        in_specs=[_AGG_SPEC, _DEG_SPEC, _W_SPEC, _B_SPEC,
                  pl.BlockSpec((D, D), lambda i: (0, 0)), _B_SPEC],
        out_specs=_H_SPEC,
        out_shape=_H_TYPE,
    )(agg, deg4, w, bbar, w_lin, b_lin)


def kernel(x, edge_index_r0, edge_index_r1, edge_index_r2,
           W_g, b_g, W_lin, b_lin):
    x_pad = jnp.pad(x.astype(_f32), ((0, N_PAD - N), (0, 0)))
    pad_idx = (N + (jnp.arange(E_PAD - E, dtype=jnp.int32) % 64))
    pad_blk = jnp.stack([pad_idx, pad_idx])
    z2 = jnp.zeros((ZTR, D), _f32)
    z1 = jnp.zeros((ROWS_PER_TILE,), _f32)

    def prep(e):
        pe = jnp.concatenate([e.astype(jnp.int32), pad_blk], axis=1)
        return pe.reshape(2, 16, NBLK, BLK)

    edges = jnp.stack([prep(edge_index_r0), prep(edge_index_r1),
                       prep(edge_index_r2)])

    deg = _deg_call(edges, z1)
    deg4 = deg.reshape(2, R, N_PAD, 1)

    hq = _prologue_call(x_pad, deg4)
    for l in range(L - 1):
        agg = _spmv_call(hq[0], hq[1], hq[2], edges, z2)
        bbar = jnp.mean(b_g[l], axis=0).reshape(1, D)
        hq = _layer_call(agg, deg4, W_g[l], bbar)

    agg = _spmv_call(hq[0], hq[1], hq[2], edges, z2)
    bbar = jnp.mean(b_g[L - 1], axis=0).reshape(1, D)
    out = _final_call(agg, deg4, W_g[L - 1], bbar, W_lin,
                      b_lin.reshape(1, D))
    return out[:N]


# speedup vs baseline: 1.3434x; 1.0132x over previous
import jax
import jax.numpy as jnp
from jax import lax
from jax.experimental import pallas as pl
from jax.experimental.pallas import tpu as pltpu
from jax.experimental.pallas import tpu_sc as plsc

N = 50000
D = 128
R = 3
E = 200000
L = 5

N_PAD = 51200
ROWS_PER_TILE = N_PAD // 16
E_PAD = 200704
BLK = 128
NBLK = 98
KFL = 2
GRP = NBLK // KFL
P_BKT = 8832
NBKT = 6
A_ROWS = NBKT * P_BKT
SP_ROWS = P_BKT + 128
ZTR = SP_ROWS // 16
BN = 512
GRID = N_PAD // BN

_f32 = jnp.float32


def _mesh():
    return plsc.VectorSubcoreMesh(core_axis_name="c", subcore_axis_name="s")


def _deg_body(edges, z1, deg, spd, idxb, ones_b):
    c = lax.axis_index("c")
    s = lax.axis_index("s")
    o16 = jnp.ones((16,), _f32)
    for i in range(BLK // 16):
        ones_b[pl.ds(i * 16, 16)] = o16

    for k in range(R):
        pltpu.sync_copy(z1, spd.at[pl.ds(s * ROWS_PER_TILE, ROWS_PER_TILE)])
        plsc.subcore_barrier()
        pltpu.sync_copy(edges.at[k, c, s], idxb)

        def sc_body(j, carry):
            pltpu.sync_copy(ones_b, spd.at[idxb.at[j]], add=True)
            return carry

        lax.fori_loop(0, NBLK, sc_body, 0)
        plsc.subcore_barrier()
        off = (c * R + k) * N_PAD + s * ROWS_PER_TILE
        pltpu.sync_copy(
            spd.at[pl.ds(s * ROWS_PER_TILE, ROWS_PER_TILE)],
            deg.at[pl.ds(off, ROWS_PER_TILE)],
        )
        plsc.subcore_barrier()


def _deg_call(edges, z1):
    return pl.kernel(
        _deg_body,
        out_type=jax.ShapeDtypeStruct((2 * R * N_PAD,), _f32),
        mesh=_mesh(),
        scratch_types=[
            pltpu.VMEM_SHARED((N_PAD,), _f32),
            pltpu.VMEM((NBLK, BLK), jnp.int32),
            pltpu.VMEM((BLK,), _f32),
        ],
    )(edges, z1)


def _spmv_body(h0, h1, h2, edges, z2, agg, spa, isrc, idst, bufs, gsem, ssem):
    c = lax.axis_index("c")
    s = lax.axis_index("s")
    lanes = lax.iota(jnp.int32, 16)

    for r in range(R):
        h_r = (h0, h1, h2)[r]
        pltpu.sync_copy(edges.at[r, 0, s], isrc)
        for p in range(NBKT // 2):
            bkt = 2 * p + c
            base = bkt * P_BKT
            pltpu.sync_copy(edges.at[r, 1, s], idst)

            def remap(j, carry):
                for kk in range(8):
                    v = idst[j, pl.ds(kk * 16, 16)]
                    rel = v - base
                    ok = (rel >= 0) & (rel < P_BKT)
                    idst[j, pl.ds(kk * 16, 16)] = jnp.where(
                        ok, rel, P_BKT + lanes)
                return carry

            lax.fori_loop(0, NBLK, remap, 0)

            pltpu.sync_copy(z2, spa.at[pl.ds(s * ZTR, ZTR)])
            plsc.subcore_barrier()

            def g_body(g, carry):
                hs = []
                for b in range(KFL):
                    j = g * KFL + b

                    @pl.when(g > 0)
                    def _drain():
                        pltpu.make_async_copy(
                            bufs.at[b], spa.at[idst.at[j]], ssem[b]).wait()

                    hs.append(pltpu.async_copy(
                        h_r.at[isrc.at[j]], bufs.at[b], gsem))
                for b in range(KFL):
                    hs[b].wait()
                for b in range(KFL):
                    j = g * KFL + b
                    pltpu.async_copy(
                        bufs.at[b], spa.at[idst.at[j]], ssem[b], add=True)
                return carry

            lax.fori_loop(0, GRP, g_body, 0)
            for b in range(KFL):
                pltpu.make_async_copy(
                    bufs.at[b], spa.at[idst.at[0]], ssem[b]).wait()
            plsc.subcore_barrier()
            nwb = P_BKT // 16
            pltpu.sync_copy(
                spa.at[pl.ds(s * nwb, nwb)],
                agg.at[r, pl.ds(base + s * nwb, nwb)],
            )
            plsc.subcore_barrier()


def _spmv_call(h0, h1, h2, edges, z2):
    return pl.kernel(
        _spmv_body,
        out_type=jax.ShapeDtypeStruct((R, A_ROWS, D), _f32),
        mesh=_mesh(),
        scratch_types=[
            pltpu.VMEM_SHARED((SP_ROWS, D), _f32),
            pltpu.VMEM((NBLK, BLK), jnp.int32),
            pltpu.VMEM((NBLK, BLK), jnp.int32),
            pltpu.VMEM((KFL, BLK, D), _f32),
            pltpu.SemaphoreType.DMA,
            [pltpu.SemaphoreType.DMA] * KFL,
        ],
    )(h0, h1, h2, edges, z2)


def _cs(deg_ref, r):
    return lax.rsqrt(jnp.maximum(deg_ref[0, r], 1.0))


def _cd3(deg_ref, r):
    return lax.rsqrt(jnp.maximum(deg_ref[1, r], 1.0)) * (1.0 / 3.0)


def _prologue_kernel(x_ref, deg_ref, o0, o1, o2):
    xv = x_ref[...]
    for r, o in enumerate((o0, o1, o2)):
        o[...] = xv * _cs(deg_ref, r)


def _accum(agg_ref, deg_ref, w_ref, b_ref):
    acc = jnp.broadcast_to(b_ref[...], (BN, D)).astype(_f32)
    wv = w_ref[...]
    for r in range(R):
        a = agg_ref[r] * _cd3(deg_ref, r)
        acc = acc + jnp.dot(a, wv[r], preferred_element_type=_f32)
    return acc


def _layer_kernel(agg_ref, deg_ref, w_ref, b_ref, o0, o1, o2):
    acc = _accum(agg_ref, deg_ref, w_ref, b_ref)
    h = jnp.where(acc >= 0, acc, 0.01 * acc)
    for r, o in enumerate((o0, o1, o2)):
        o[...] = h * _cs(deg_ref, r)


def _final_kernel(agg_ref, deg_ref, w_ref, b_ref, wl_ref, bl_ref, out_ref):
    h = _accum(agg_ref, deg_ref, w_ref, b_ref)
    out_ref[...] = (
        jnp.dot(h, wl_ref[...], preferred_element_type=_f32) + bl_ref[...]
    )


_DEG_SPEC = pl.BlockSpec((2, R, BN, 1), lambda i: (0, 0, i, 0))
_AGG_SPEC = pl.BlockSpec((R, BN, D), lambda i: (0, i, 0))
_W_SPEC = pl.BlockSpec((R, D, D), lambda i: (0, 0, 0))
_B_SPEC = pl.BlockSpec((1, D), lambda i: (0, 0))
_H_SPEC = pl.BlockSpec((BN, D), lambda i: (i, 0))
_H_TYPE = jax.ShapeDtypeStruct((N_PAD, D), _f32)


def _prologue_call(x_pad, deg4):
    return pl.pallas_call(
        _prologue_kernel,
        grid=(GRID,),
        in_specs=[_H_SPEC, _DEG_SPEC],
        out_specs=[_H_SPEC] * 3,
        out_shape=[_H_TYPE] * 3,
    )(x_pad, deg4)


def _layer_call(agg, deg4, w, bbar):
    return pl.pallas_call(
        _layer_kernel,
        grid=(GRID,),
        in_specs=[_AGG_SPEC, _DEG_SPEC, _W_SPEC, _B_SPEC],
        out_specs=[_H_SPEC] * 3,
        out_shape=[_H_TYPE] * 3,
    )(agg, deg4, w, bbar)


def _final_call(agg, deg4, w, bbar, w_lin, b_lin):
    return pl.pallas_call(
        _final_kernel,
        grid=(GRID,),
        in_specs=[_AGG_SPEC, _DEG_SPEC, _W_SPEC, _B_SPEC,
                  pl.BlockSpec((D, D), lambda i: (0, 0)), _B_SPEC],
        out_specs=_H_SPEC,
        out_shape=_H_TYPE,
    )(agg, deg4, w, bbar, w_lin, b_lin)


def kernel(x, edge_index_r0, edge_index_r1, edge_index_r2,
           W_g, b_g, W_lin, b_lin):
    x_pad = jnp.pad(x.astype(_f32), ((0, N_PAD - N), (0, 0)))
    pad_idx = (N + (jnp.arange(E_PAD - E, dtype=jnp.int32) % 64))
    pad_blk = jnp.stack([pad_idx, pad_idx])
    z2 = jnp.zeros((ZTR, D), _f32)
    z1 = jnp.zeros((ROWS_PER_TILE,), _f32)

    def prep(e):
        pe = jnp.concatenate([e.astype(jnp.int32), pad_blk], axis=1)
        return pe.reshape(2, 16, NBLK, BLK)

    edges = jnp.stack([prep(edge_index_r0), prep(edge_index_r1),
                       prep(edge_index_r2)])

    deg = _deg_call(edges, z1)
    deg4 = deg.reshape(2, R, N_PAD, 1)

    hq = _prologue_call(x_pad, deg4)
    for l in range(L - 1):
        agg = _spmv_call(hq[0], hq[1], hq[2], edges, z2)
        bbar = jnp.mean(b_g[l], axis=0).reshape(1, D)
        hq = _layer_call(agg, deg4, W_g[l], bbar)

    agg = _spmv_call(hq[0], hq[1], hq[2], edges, z2)
    bbar = jnp.mean(b_g[L - 1], axis=0).reshape(1, D)
    out = _final_call(agg, deg4, W_g[L - 1], bbar, W_lin,
                      b_lin.reshape(1, D))
    return out[:N]
